# Initial kernel scaffold; baseline (speedup 1.0000x reference)
import jax
import jax.numpy as jnp
from jax.experimental import pallas as pl


def kernel(x, position_encoding):
    raise NotImplementedError("write your pallas kernel here")


# speedup vs baseline: 7.9335x; 7.9335x over previous
import functools

import jax
import jax.numpy as jnp
from jax import lax
from jax.experimental import pallas as pl
from jax.experimental.pallas import tpu as pltpu
from jax.experimental.pallas import tpu_sc as plsc

NUM_TERMS = 16
NUM_COLS = 16384
DEPTH = 128
NUM_WORKERS = 32
COLS_PER_W = NUM_COLS // NUM_WORKERS
CHUNK = 128
NUM_CHUNKS = COLS_PER_W // CHUNK
LANES = 16


def _body(x_hbm, pe_hbm, out_hbm, idx_v, acc_v, sem):
    cid = lax.axis_index("c")
    sid = lax.axis_index("s")
    wid = sid * 2 + cid
    inv = jnp.float32(1.0 / NUM_TERMS)

    def chunk_body(k, carry):
        base = wid * COLS_PER_W + k * CHUNK
        pltpu.sync_copy(x_hbm.at[:, pl.ds(base, CHUNK)], idx_v)
        pltpu.sync_copy(pe_hbm.at[idx_v.at[0]], acc_v)
        for i in range(1, NUM_TERMS):
            pltpu.sync_copy(pe_hbm.at[idx_v.at[i]], acc_v, add=True)

        def row_body(r, c2):
            for j in range(DEPTH // LANES):
                sl = pl.ds(j * LANES, LANES)
                acc_v[r, sl] = acc_v[r, sl] * inv
            return c2

        lax.fori_loop(0, CHUNK, row_body, 0)
        pltpu.sync_copy(acc_v, out_hbm.at[pl.ds(base, CHUNK)])
        return carry

    lax.fori_loop(0, NUM_CHUNKS, chunk_body, 0)


@jax.jit
def kernel(x, position_encoding):
    mesh = plsc.VectorSubcoreMesh(core_axis_name="c", subcore_axis_name="s")
    f = pl.kernel(
        _body,
        mesh=mesh,
        out_type=jax.ShapeDtypeStruct((NUM_COLS, DEPTH), jnp.float32),
        scratch_types=[
            pltpu.VMEM((NUM_TERMS, CHUNK), jnp.int32),
            pltpu.VMEM((CHUNK, DEPTH), jnp.float32),
            pltpu.SemaphoreType.DMA,
        ],
    )
    return f(x, position_encoding)


# speedup vs baseline: 12.6629x; 1.5961x over previous
import jax
import jax.numpy as jnp
from jax import lax
from jax.experimental import pallas as pl
from jax.experimental.pallas import tpu as pltpu
from jax.experimental.pallas import tpu_sc as plsc

NUM_TERMS = 16
NUM_COLS = 16384
DEPTH = 128
NUM_WORKERS = 32
COLS_PER_W = NUM_COLS // NUM_WORKERS
CHUNK = 128
NUM_CHUNKS = COLS_PER_W // CHUNK
LANES = 16
VECS_PER_ROW = DEPTH // LANES


def _body(x_hbm, pe_hbm, out_hbm,
          idx_v, acc0, acc1, stage0, stage1,
          gsem0, gsem1, wsem0, wsem1):
    cid = lax.axis_index("c")
    sid = lax.axis_index("s")
    wid = sid * 2 + cid
    col0 = wid * COLS_PER_W
    inv = jnp.float32(1.0 / NUM_TERMS)
    zvec = jnp.zeros((LANES,), jnp.float32)

    accs = (acc0, acc1)
    stages = (stage0, stage1)
    gsems = (gsem0, gsem1)
    wsems = (wsem0, wsem1)

    pltpu.sync_copy(x_hbm.at[:, pl.ds(col0, COLS_PER_W)], idx_v)

    def zero_body(r, carry):
        for j in range(VECS_PER_ROW):
            sl = pl.ds(j * LANES, LANES)
            acc0[r, sl] = zvec
            acc1[r, sl] = zvec
        return carry

    lax.fori_loop(0, CHUNK, zero_body, 0)

    def fire_gathers(k):
        acc, gsem = accs[k % 2], gsems[k % 2]
        cds = []
        for i in range(NUM_TERMS):
            idx = idx_v.at[i, pl.ds(k * CHUNK, CHUNK)]
            cds.append(pltpu.async_copy(pe_hbm.at[idx], acc, gsem, add=True))
        return cds

    def scale_and_zero(k):
        acc, stage = accs[k % 2], stages[k % 2]

        def row_body(r, carry):
            for j in range(VECS_PER_ROW):
                sl = pl.ds(j * LANES, LANES)
                stage[r, sl] = acc[r, sl] * inv
                acc[r, sl] = zvec
            return carry

        lax.fori_loop(0, CHUNK, row_body, 0)

    pending = {0: fire_gathers(0), 1: fire_gathers(1)}
    wb = {}
    for k in range(NUM_CHUNKS):
        b = k % 2
        for cd in pending.pop(k):
            cd.wait()
        if k - 2 in wb:
            wb.pop(k - 2).wait()
        scale_and_zero(k)
        wb[k] = pltpu.async_copy(
            stages[b], out_hbm.at[pl.ds(col0 + k * CHUNK, CHUNK)], wsems[b])
        if k + 2 < NUM_CHUNKS:
            pending[k + 2] = fire_gathers(k + 2)
    for k in sorted(wb):
        wb.pop(k).wait()


@jax.jit
def kernel(x, position_encoding):
    mesh = plsc.VectorSubcoreMesh(core_axis_name="c", subcore_axis_name="s")
    f = pl.kernel(
        _body,
        mesh=mesh,
        out_type=jax.ShapeDtypeStruct((NUM_COLS, DEPTH), jnp.float32),
        scratch_types=[
            pltpu.VMEM((NUM_TERMS, COLS_PER_W), jnp.int32),
            pltpu.VMEM((CHUNK, DEPTH), jnp.float32),
            pltpu.VMEM((CHUNK, DEPTH), jnp.float32),
            pltpu.VMEM((CHUNK, DEPTH), jnp.float32),
            pltpu.VMEM((CHUNK, DEPTH), jnp.float32),
            pltpu.SemaphoreType.DMA,
            pltpu.SemaphoreType.DMA,
            pltpu.SemaphoreType.DMA,
            pltpu.SemaphoreType.DMA,
        ],
    )
    return f(x, position_encoding)


# speedup vs baseline: 12.7881x; 1.0099x over previous
import jax
import jax.numpy as jnp
from jax import lax
from jax.experimental import pallas as pl
from jax.experimental.pallas import tpu as pltpu
from jax.experimental.pallas import tpu_sc as plsc

NUM_TERMS = 16
NUM_COLS = 16384
DEPTH = 128
NUM_WORKERS = 32
COLS_PER_W = NUM_COLS // NUM_WORKERS
CHUNK = 128
NUM_CHUNKS = COLS_PER_W // CHUNK
LANES = 16
VECS_PER_ROW = DEPTH // LANES
DEPTH_G = 3


def _body(x_hbm, pe_hbm, out_hbm,
          idx_v, acc0, acc1, acc2, stage0, stage1,
          gsem0, gsem1, gsem2, wsem0, wsem1):
    cid = lax.axis_index("c")
    sid = lax.axis_index("s")
    wid = sid * 2 + cid
    col0 = wid * COLS_PER_W
    inv = jnp.float32(1.0 / NUM_TERMS)
    zvec = jnp.zeros((LANES,), jnp.float32)

    accs = (acc0, acc1, acc2)
    stages = (stage0, stage1)
    gsems = (gsem0, gsem1, gsem2)
    wsems = (wsem0, wsem1)

    def zero_acc(acc):
        def zbody(r, carry):
            for j in range(VECS_PER_ROW):
                acc[r, pl.ds(j * LANES, LANES)] = zvec
            return carry
        lax.fori_loop(0, CHUNK, zbody, 0)

    def fire(k):
        acc, gsem = accs[k % DEPTH_G], gsems[k % DEPTH_G]
        return [
            pltpu.async_copy(
                pe_hbm.at[idx_v.at[i, pl.ds(k * CHUNK, CHUNK)]],
                acc, gsem, add=True)
            for i in range(NUM_TERMS)
        ]

    zero_acc(acc0)
    pltpu.sync_copy(x_hbm.at[:, pl.ds(col0, CHUNK)], idx_v.at[:, pl.ds(0, CHUNK)])
    pending = {0: fire(0)}
    pltpu.sync_copy(x_hbm.at[:, pl.ds(col0 + CHUNK, COLS_PER_W - CHUNK)],
                    idx_v.at[:, pl.ds(CHUNK, COLS_PER_W - CHUNK)])
    zero_acc(acc1)
    pending[1] = fire(1)
    zero_acc(acc2)
    pending[2] = fire(2)

    wb = {}
    for k in range(NUM_CHUNKS):
        acc, stage = accs[k % DEPTH_G], stages[k % 2]
        for cd in pending.pop(k):
            cd.wait()
        if k - 2 in wb:
            wb.pop(k - 2).wait()

        def row_body(r2, carry):
            for r in (2 * r2, 2 * r2 + 1):
                for j in range(VECS_PER_ROW):
                    sl = pl.ds(j * LANES, LANES)
                    stage[r, sl] = acc[r, sl] * inv
                    acc[r, sl] = zvec
            return carry

        lax.fori_loop(0, CHUNK // 2, row_body, 0)
        wb[k] = pltpu.async_copy(
            stage, out_hbm.at[pl.ds(col0 + k * CHUNK, CHUNK)], wsems[k % 2])
        if k + DEPTH_G < NUM_CHUNKS:
            pending[k + DEPTH_G] = fire(k + DEPTH_G)
    for k in sorted(wb):
        wb.pop(k).wait()


@jax.jit
def kernel(x, position_encoding):
    mesh = plsc.VectorSubcoreMesh(core_axis_name="c", subcore_axis_name="s")
    f = pl.kernel(
        _body,
        mesh=mesh,
        out_type=jax.ShapeDtypeStruct((NUM_COLS, DEPTH), jnp.float32),
        scratch_types=[
            pltpu.VMEM((NUM_TERMS, COLS_PER_W), jnp.int32),
            pltpu.VMEM((CHUNK, DEPTH), jnp.float32),
            pltpu.VMEM((CHUNK, DEPTH), jnp.float32),
            pltpu.VMEM((CHUNK, DEPTH), jnp.float32),
            pltpu.VMEM((CHUNK, DEPTH), jnp.float32),
            pltpu.VMEM((CHUNK, DEPTH), jnp.float32),
            pltpu.SemaphoreType.DMA,
            pltpu.SemaphoreType.DMA,
            pltpu.SemaphoreType.DMA,
            pltpu.SemaphoreType.DMA,
            pltpu.SemaphoreType.DMA,
        ],
    )
    return f(x, position_encoding)


# speedup vs baseline: 13.0685x; 1.0219x over previous
import jax
import jax.numpy as jnp
from jax import lax
from jax.experimental import pallas as pl
from jax.experimental.pallas import tpu as pltpu
from jax.experimental.pallas import tpu_sc as plsc

NUM_TERMS = 16
NUM_COLS = 16384
DEPTH = 128
NUM_WORKERS = 32
COLS_PER_W = NUM_COLS // NUM_WORKERS
CHUNK = 128
NUM_CHUNKS = COLS_PER_W // CHUNK
LANES = 16
VECS_PER_ROW = DEPTH // LANES


def _body(x_hbm, pe_hbm, out_hbm,
          idx_v, acc0, acc1, acc2, acc3, stage0, stage1,
          gsem0, gsem1, gsem2, gsem3, wsem0, wsem1):
    cid = lax.axis_index("c")
    sid = lax.axis_index("s")
    wid = sid * 2 + cid
    col0 = wid * COLS_PER_W
    inv = jnp.float32(1.0 / NUM_TERMS)
    zvec = jnp.zeros((LANES,), jnp.float32)

    accs = (acc0, acc1, acc2, acc3)
    stages = (stage0, stage1)
    gsems = (gsem0, gsem1, gsem2, gsem3)
    wsems = (wsem0, wsem1)

    def zero_acc(acc):
        def zbody(r, carry):
            for j in range(VECS_PER_ROW):
                acc[r, pl.ds(j * LANES, LANES)] = zvec
            return carry
        lax.fori_loop(0, CHUNK, zbody, 0)

    def fire(k):
        return [
            pltpu.async_copy(
                pe_hbm.at[idx_v.at[i, pl.ds(k * CHUNK, CHUNK)]],
                accs[k], gsems[k], add=True)
            for i in range(NUM_TERMS)
        ]

    zero_acc(acc0)
    pltpu.sync_copy(x_hbm.at[:, pl.ds(col0, CHUNK)], idx_v.at[:, pl.ds(0, CHUNK)])
    pending = {0: fire(0)}
    pltpu.sync_copy(x_hbm.at[:, pl.ds(col0 + CHUNK, COLS_PER_W - CHUNK)],
                    idx_v.at[:, pl.ds(CHUNK, COLS_PER_W - CHUNK)])
    for k in range(1, NUM_CHUNKS):
        zero_acc(accs[k])
        pending[k] = fire(k)

    wb = {}
    for k in range(NUM_CHUNKS):
        acc, stage = accs[k], stages[k % 2]
        for cd in pending.pop(k):
            cd.wait()
        if k - 2 in wb:
            wb.pop(k - 2).wait()

        def row_body(r2, carry):
            for r in (2 * r2, 2 * r2 + 1):
                for j in range(VECS_PER_ROW):
                    sl = pl.ds(j * LANES, LANES)
                    stage[r, sl] = acc[r, sl] * inv
            return carry

        lax.fori_loop(0, CHUNK // 2, row_body, 0)
        wb[k] = pltpu.async_copy(
            stage, out_hbm.at[pl.ds(col0 + k * CHUNK, CHUNK)], wsems[k % 2])
    for k in sorted(wb):
        wb.pop(k).wait()


@jax.jit
def kernel(x, position_encoding):
    mesh = plsc.VectorSubcoreMesh(core_axis_name="c", subcore_axis_name="s")
    f = pl.kernel(
        _body,
        mesh=mesh,
        out_type=jax.ShapeDtypeStruct((NUM_COLS, DEPTH), jnp.float32),
        scratch_types=[
            pltpu.VMEM((NUM_TERMS, COLS_PER_W), jnp.int32),
            pltpu.VMEM((CHUNK, DEPTH), jnp.float32),
            pltpu.VMEM((CHUNK, DEPTH), jnp.float32),
            pltpu.VMEM((CHUNK, DEPTH), jnp.float32),
            pltpu.VMEM((CHUNK, DEPTH), jnp.float32),
            pltpu.VMEM((CHUNK, DEPTH), jnp.float32),
            pltpu.VMEM((CHUNK, DEPTH), jnp.float32),
            pltpu.SemaphoreType.DMA,
            pltpu.SemaphoreType.DMA,
            pltpu.SemaphoreType.DMA,
            pltpu.SemaphoreType.DMA,
            pltpu.SemaphoreType.DMA,
            pltpu.SemaphoreType.DMA,
        ],
    )
    return f(x, position_encoding)
